# baseline (speedup 1.0000x reference)
import functools

import jax
import jax.numpy as jnp
from jax.experimental import pallas as pl
from jax.experimental.pallas import tpu as pltpu

_N_PROP = 20000
_N_PAD = 20480
_ROWS = 160
_COLS = 128
_MAX_GT = 100
_GT_PAD = 128
_POS = 66
_NEG = 134
_TOT = 200
_MH = 28
_MW = 28
_HM = 512
_WM = 512
_BBOX_STD = (0.1, 0.1, 0.2, 0.2)


def _select_kernel(props_ref, y1p_ref, x1p_ref, y2p_ref, x2p_ref,
                   gtb_s_ref, ids_s_ref, gtv_ref, idv_ref,
                   rois_ref, ids_ref, deltas_ref, assign_ref, pvalid_ref):
    f32 = jnp.float32
    y1p = y1p_ref[:, :]
    x1p = x1p_ref[:, :]
    y2p = y2p_ref[:, :]
    x2p = x2p_ref[:, :]
    prop_valid = (jnp.abs(y1p) + jnp.abs(x1p) + jnp.abs(y2p) + jnp.abs(x2p)) > 0.0
    area_p = (y2p - y1p) * (x2p - x1p)

    def gt_step(g, carry):
        roi_max, crowd_max = carry
        y1g = gtb_s_ref[0, g]
        x1g = gtb_s_ref[1, g]
        y2g = gtb_s_ref[2, g]
        x2g = gtb_s_ref[3, g]
        cid = ids_s_ref[0, g]
        gt_valid = (jnp.abs(y1g) + jnp.abs(x1g) + jnp.abs(y2g) + jnp.abs(x2g)) > 0.0
        yA = jnp.maximum(y1p, y1g)
        xA = jnp.maximum(x1p, x1g)
        yB = jnp.minimum(y2p, y2g)
        xB = jnp.minimum(x2p, x2g)
        inter = jnp.maximum(yB - yA, 0.0) * jnp.maximum(xB - xA, 0.0)
        area_g = (y2g - y1g) * (x2g - x1g)
        union = area_p + area_g - inter
        iou = jnp.where(union > 0.0, inter / jnp.maximum(union, 1e-12), 0.0)
        nc = jnp.logical_and(cid > 0, gt_valid)
        cr = jnp.logical_and(cid < 0, gt_valid)
        roi_max = jnp.maximum(roi_max, jnp.where(nc, iou, 0.0))
        crowd_max = jnp.maximum(crowd_max, jnp.where(cr, iou, 0.0))
        return roi_max, crowd_max

    zeros = jnp.zeros((_ROWS, _COLS), f32)
    roi_max, crowd_max = jax.lax.fori_loop(0, _GT_PAD, gt_step, (zeros, zeros))

    no_crowd = crowd_max < 0.001
    pos_scores = jnp.where(jnp.logical_and(roi_max >= 0.5, prop_valid),
                           roi_max, -1.0)
    neg_scores = jnp.where(
        jnp.logical_and(jnp.logical_and(roi_max < 0.5, no_crowd), prop_valid),
        1.0 - roi_max, -1.0)

    r_iota = jax.lax.broadcasted_iota(jnp.int32, (_ROWS, _COLS), 0)
    c_iota = jax.lax.broadcasted_iota(jnp.int32, (_ROWS, _COLS), 1)
    flat_iota = r_iota * _COLS + c_iota
    g_iota = jax.lax.broadcasted_iota(jnp.int32, (1, _GT_PAD), 1)
    lane4 = jax.lax.broadcasted_iota(jnp.int32, (1, 4), 1)
    std_row = jnp.where(lane4 < 2, jnp.float32(_BBOX_STD[0]),
                        jnp.float32(_BBOX_STD[2]))

    gy1 = gtv_ref[0:1, :]
    gx1 = gtv_ref[1:2, :]
    gy2 = gtv_ref[2:3, :]
    gx2 = gtv_ref[3:4, :]
    gids = idv_ref[0:1, :]
    g_valid = (jnp.abs(gy1) + jnp.abs(gx1) + jnp.abs(gy2) + jnp.abs(gx2)) > 0.0
    g_nc = jnp.logical_and(gids > 0, g_valid)

    def pos_step(i, scores):
        m = jnp.max(scores)
        idx = jnp.min(jnp.where(scores == m, flat_iota, jnp.int32(2 ** 30)))
        valid = m > 0.0
        vf = valid.astype(f32)
        row = props_ref[pl.ds(idx, 1), :]
        rois_ref[pl.ds(i, 1), :] = row * vf
        py1 = row[0, 0]
        px1 = row[0, 1]
        py2 = row[0, 2]
        px2 = row[0, 3]
        yA = jnp.maximum(py1, gy1)
        xA = jnp.maximum(px1, gx1)
        yB = jnp.minimum(py2, gy2)
        xB = jnp.minimum(px2, gx2)
        inter = jnp.maximum(yB - yA, 0.0) * jnp.maximum(xB - xA, 0.0)
        a_p = (py2 - py1) * (px2 - px1)
        a_g = (gy2 - gy1) * (gx2 - gx1)
        union = a_p + a_g - inter
        iou = jnp.where(union > 0.0, inter / jnp.maximum(union, 1e-12), 0.0)
        ov_nc = jnp.where(g_nc, iou, 0.0)
        mx = jnp.max(ov_nc)
        asn = jnp.min(jnp.where(ov_nc == mx, g_iota, jnp.int32(2 ** 30)))
        sel = g_iota == asn
        by1 = jnp.sum(jnp.where(sel, gy1, 0.0))
        bx1 = jnp.sum(jnp.where(sel, gx1, 0.0))
        by2 = jnp.sum(jnp.where(sel, gy2, 0.0))
        bx2 = jnp.sum(jnp.where(sel, gx2, 0.0))
        cid = jnp.sum(jnp.where(sel, gids, 0))
        rh = jnp.maximum((py2 - py1) * vf, 1e-8)
        rw = jnp.maximum((px2 - px1) * vf, 1e-8)
        rcy = py1 * vf + 0.5 * rh
        rcx = px1 * vf + 0.5 * rw
        gh = jnp.maximum(by2 - by1, 1e-8)
        gw = jnp.maximum(bx2 - bx1, 1e-8)
        gcy = by1 + 0.5 * gh
        gcx = bx1 + 0.5 * gw
        logs = jnp.log(jnp.where(lane4 == 2, gh / rh,
                                 jnp.where(lane4 == 3, gw / rw, 1.0)))
        dyv = (gcy - rcy) / rh
        dxv = (gcx - rcx) / rw
        drow = jnp.where(lane4 == 0, dyv,
                         jnp.where(lane4 == 1, dxv, logs))
        drow = (drow / std_row) * vf
        deltas_ref[pl.ds(i, 1), :] = drow
        ids_ref[pl.ds(i, 1), :] = jnp.full((1, 1), cid * valid.astype(jnp.int32),
                                           jnp.int32)
        assign_ref[pl.ds(i, 1), :] = jnp.full((1, 1), asn, jnp.int32)
        pvalid_ref[pl.ds(i, 1), :] = jnp.full((1, 1), vf, f32)
        return jnp.where(flat_iota == idx, -jnp.inf, scores)

    jax.lax.fori_loop(0, _POS, pos_step, pos_scores)

    def neg_step(i, scores):
        m = jnp.max(scores)
        idx = jnp.min(jnp.where(scores == m, flat_iota, jnp.int32(2 ** 30)))
        vf = (m > 0.0).astype(f32)
        row = props_ref[pl.ds(idx, 1), :]
        j = i + _POS
        rois_ref[pl.ds(j, 1), :] = row * vf
        deltas_ref[pl.ds(j, 1), :] = jnp.zeros((1, 4), f32)
        ids_ref[pl.ds(j, 1), :] = jnp.zeros((1, 1), jnp.int32)
        assign_ref[pl.ds(j, 1), :] = jnp.zeros((1, 1), jnp.int32)
        pvalid_ref[pl.ds(j, 1), :] = jnp.zeros((1, 1), f32)
        return jnp.where(flat_iota == idx, -jnp.inf, scores)

    jax.lax.fori_loop(0, _NEG, neg_step, neg_scores)


def _crop_kernel(assign_s_ref, mask_ref, rois_s_ref, pv_s_ref, out_ref):
    f32 = jnp.float32
    i = pl.program_id(0)
    m = mask_ref[0].astype(f32)
    y1 = rois_s_ref[i, 0]
    x1 = rois_s_ref[i, 1]
    y2 = rois_s_ref[i, 2]
    x2 = rois_s_ref[i, 3]
    pv = pv_s_ref[i]

    iy = jax.lax.broadcasted_iota(jnp.int32, (_MH, 1), 0).astype(f32)
    ys = y1 * (_HM - 1) + iy * ((y2 - y1) * (_HM - 1) / (_MH - 1))
    y0 = jnp.floor(ys)
    y0i = jnp.clip(y0, 0, _HM - 1).astype(jnp.int32)
    y1i = jnp.clip(y0 + 1, 0, _HM - 1).astype(jnp.int32)
    wy = ys - y0

    ixc = jax.lax.broadcasted_iota(jnp.int32, (_MW, 1), 0).astype(f32)
    xs_c = x1 * (_WM - 1) + ixc * ((x2 - x1) * (_WM - 1) / (_MW - 1))
    x0_c = jnp.floor(xs_c)
    x0i = jnp.clip(x0_c, 0, _WM - 1).astype(jnp.int32)
    x1i = jnp.clip(x0_c + 1, 0, _WM - 1).astype(jnp.int32)

    ixl = jax.lax.broadcasted_iota(jnp.int32, (1, _MW), 1).astype(f32)
    xs_l = x1 * (_WM - 1) + ixl * ((x2 - x1) * (_WM - 1) / (_MW - 1))
    wx = xs_l - jnp.floor(xs_l)

    ycat = jnp.concatenate([y0i, y1i], axis=0)
    xcat = jnp.concatenate([x0i, x1i], axis=0)
    lane_h = jax.lax.broadcasted_iota(jnp.int32, (2 * _MH, _HM), 1)
    Ycat = (lane_h == ycat).astype(f32)
    lane_w = jax.lax.broadcasted_iota(jnp.int32, (2 * _MW, _WM), 1)
    Xcat = (lane_w == xcat).astype(f32)

    dn = (((1,), (1,)), ((), ()))
    dnm = (((1,), (0,)), ((), ()))
    Acat = jax.lax.dot_general(Ycat, m, dnm, preferred_element_type=f32)
    G = jax.lax.dot_general(Acat, Xcat, dn, preferred_element_type=f32)
    g00 = G[:_MH, :_MW]
    g01 = G[:_MH, _MW:]
    g10 = G[_MH:, :_MW]
    g11 = G[_MH:, _MW:]

    val = (((1.0 - wy) * (1.0 - wx)) * g00
           + ((1.0 - wy) * wx) * g01
           + (wy * (1.0 - wx)) * g10
           + (wy * wx) * g11)
    in_y = jnp.logical_and(ys >= 0.0, ys <= (_HM - 1))
    in_x = jnp.logical_and(xs_l >= 0.0, xs_l <= (_WM - 1))
    val = jnp.where(jnp.logical_and(in_y, in_x), val, 0.0)
    out_ref[0] = jnp.round(val) * pv


@jax.jit
def kernel(proposals, prior_class_ids, prior_boxes, prior_masks):
    f32 = jnp.float32
    props = proposals[0]
    props_pad = jnp.pad(props, ((0, _N_PAD - _N_PROP), (0, 0)))
    coords = jnp.reshape(jnp.transpose(props_pad), (4, _ROWS, _COLS))
    y1p, x1p, y2p, x2p = coords[0], coords[1], coords[2], coords[3]

    gtb = jnp.pad(prior_boxes[0], ((0, _GT_PAD - _MAX_GT), (0, 0)))
    gtb_t = jnp.transpose(gtb)
    ids_row = jnp.pad(prior_class_ids[0], (0, _GT_PAD - _MAX_GT))[None, :]

    smem = pl.BlockSpec(memory_space=pltpu.SMEM)
    rois, out_ids, deltas, assign, pvalid = pl.pallas_call(
        _select_kernel,
        in_specs=[pl.BlockSpec(props_pad.shape, lambda: (0, 0)),
                  pl.BlockSpec((_ROWS, _COLS), lambda: (0, 0)),
                  pl.BlockSpec((_ROWS, _COLS), lambda: (0, 0)),
                  pl.BlockSpec((_ROWS, _COLS), lambda: (0, 0)),
                  pl.BlockSpec((_ROWS, _COLS), lambda: (0, 0)),
                  smem, smem,
                  pl.BlockSpec((4, _GT_PAD), lambda: (0, 0)),
                  pl.BlockSpec((1, _GT_PAD), lambda: (0, 0))],
        out_specs=[pl.BlockSpec((_TOT, 4), lambda: (0, 0)),
                   pl.BlockSpec((_TOT, 1), lambda: (0, 0)),
                   pl.BlockSpec((_TOT, 4), lambda: (0, 0)),
                   pl.BlockSpec((_TOT, 1), lambda: (0, 0)),
                   pl.BlockSpec((_TOT, 1), lambda: (0, 0))],
        out_shape=[jax.ShapeDtypeStruct((_TOT, 4), f32),
                   jax.ShapeDtypeStruct((_TOT, 1), jnp.int32),
                   jax.ShapeDtypeStruct((_TOT, 4), f32),
                   jax.ShapeDtypeStruct((_TOT, 1), jnp.int32),
                   jax.ShapeDtypeStruct((_TOT, 1), f32)],
    )(props_pad, y1p, x1p, y2p, x2p, gtb_t, ids_row, gtb_t, ids_row)

    masks_t = jnp.transpose(prior_masks[0], (2, 0, 1)).astype(jnp.int8)
    assign_flat = assign[:_POS, 0]
    pos_rois = rois[:_POS]
    pv_flat = pvalid[:_POS, 0]

    grid_spec = pltpu.PrefetchScalarGridSpec(
        num_scalar_prefetch=1,
        grid=(_POS,),
        in_specs=[pl.BlockSpec((1, _HM, _WM), lambda i, a: (a[i], 0, 0)),
                  pl.BlockSpec(memory_space=pltpu.SMEM),
                  pl.BlockSpec(memory_space=pltpu.SMEM)],
        out_specs=pl.BlockSpec((1, _MH, _MW), lambda i, a: (i, 0, 0)),
    )
    cropped = pl.pallas_call(
        _crop_kernel,
        grid_spec=grid_spec,
        out_shape=jax.ShapeDtypeStruct((_POS, _MH, _MW), f32),
    )(assign_flat, masks_t, pos_rois, pv_flat)

    out_masks = jnp.concatenate(
        [cropped, jnp.zeros((_NEG, _MH, _MW), f32)], axis=0)
    return (rois[None], out_ids[None, :, 0], deltas[None],
            out_masks[None])


# speedup vs baseline: 1.0070x; 1.0070x over previous
import functools

import jax
import jax.numpy as jnp
from jax.experimental import pallas as pl
from jax.experimental.pallas import tpu as pltpu

_N_PROP = 20000
_N_PAD = 20480
_ROWS = 160
_COLS = 128
_MAX_GT = 100
_GT_PAD = 128
_POS = 66
_NEG = 134
_TOT = 200
_MH = 28
_MW = 28
_HM = 512
_WM = 512
_BBOX_STD = (0.1, 0.1, 0.2, 0.2)


def _select_kernel(props_ref, y1p_ref, x1p_ref, y2p_ref, x2p_ref,
                   gtb_s_ref, ids_s_ref, gtv_ref, idv_ref,
                   rois_ref, ids_ref, deltas_ref, assign_ref, pvalid_ref):
    f32 = jnp.float32
    y1p = y1p_ref[:, :]
    x1p = x1p_ref[:, :]
    y2p = y2p_ref[:, :]
    x2p = x2p_ref[:, :]
    prop_valid = (jnp.abs(y1p) + jnp.abs(x1p) + jnp.abs(y2p) + jnp.abs(x2p)) > 0.0
    area_p = (y2p - y1p) * (x2p - x1p)

    def gt_step(g, carry):
        roi_max, crowd_max = carry
        y1g = gtb_s_ref[0, g]
        x1g = gtb_s_ref[1, g]
        y2g = gtb_s_ref[2, g]
        x2g = gtb_s_ref[3, g]
        cid = ids_s_ref[0, g]
        gt_valid = (jnp.abs(y1g) + jnp.abs(x1g) + jnp.abs(y2g) + jnp.abs(x2g)) > 0.0
        yA = jnp.maximum(y1p, y1g)
        xA = jnp.maximum(x1p, x1g)
        yB = jnp.minimum(y2p, y2g)
        xB = jnp.minimum(x2p, x2g)
        inter = jnp.maximum(yB - yA, 0.0) * jnp.maximum(xB - xA, 0.0)
        area_g = (y2g - y1g) * (x2g - x1g)
        union = area_p + area_g - inter
        iou = jnp.where(union > 0.0, inter / jnp.maximum(union, 1e-12), 0.0)
        nc = jnp.logical_and(cid > 0, gt_valid)
        cr = jnp.logical_and(cid < 0, gt_valid)
        roi_max = jnp.maximum(roi_max, jnp.where(nc, iou, 0.0))
        crowd_max = jnp.maximum(crowd_max, jnp.where(cr, iou, 0.0))
        return roi_max, crowd_max

    zeros = jnp.zeros((_ROWS, _COLS), f32)
    roi_max, crowd_max = jax.lax.fori_loop(0, _MAX_GT, gt_step, (zeros, zeros))

    no_crowd = crowd_max < 0.001
    pos_scores = jnp.where(jnp.logical_and(roi_max >= 0.5, prop_valid),
                           roi_max, -1.0)
    neg_scores = jnp.where(
        jnp.logical_and(jnp.logical_and(roi_max < 0.5, no_crowd), prop_valid),
        1.0 - roi_max, -1.0)

    r_iota = jax.lax.broadcasted_iota(jnp.int32, (_ROWS, _COLS), 0)
    c_iota = jax.lax.broadcasted_iota(jnp.int32, (_ROWS, _COLS), 1)
    flat_iota = r_iota * _COLS + c_iota
    g_iota = jax.lax.broadcasted_iota(jnp.int32, (1, _GT_PAD), 1)
    lane4 = jax.lax.broadcasted_iota(jnp.int32, (1, 4), 1)
    std_row = jnp.where(lane4 < 2, jnp.float32(_BBOX_STD[0]),
                        jnp.float32(_BBOX_STD[2]))

    gy1 = gtv_ref[0:1, :]
    gx1 = gtv_ref[1:2, :]
    gy2 = gtv_ref[2:3, :]
    gx2 = gtv_ref[3:4, :]
    gids = idv_ref[0:1, :]
    g_valid = (jnp.abs(gy1) + jnp.abs(gx1) + jnp.abs(gy2) + jnp.abs(gx2)) > 0.0
    g_nc = jnp.logical_and(gids > 0, g_valid)

    def pos_step(i, scores):
        m = jnp.max(scores)
        idx = jnp.min(jnp.where(scores == m, flat_iota, jnp.int32(2 ** 30)))
        valid = m > 0.0
        vf = valid.astype(f32)
        row = props_ref[pl.ds(idx, 1), :]
        rois_ref[pl.ds(i, 1), :] = row * vf
        py1 = row[0, 0]
        px1 = row[0, 1]
        py2 = row[0, 2]
        px2 = row[0, 3]
        yA = jnp.maximum(py1, gy1)
        xA = jnp.maximum(px1, gx1)
        yB = jnp.minimum(py2, gy2)
        xB = jnp.minimum(px2, gx2)
        inter = jnp.maximum(yB - yA, 0.0) * jnp.maximum(xB - xA, 0.0)
        a_p = (py2 - py1) * (px2 - px1)
        a_g = (gy2 - gy1) * (gx2 - gx1)
        union = a_p + a_g - inter
        iou = jnp.where(union > 0.0, inter / jnp.maximum(union, 1e-12), 0.0)
        ov_nc = jnp.where(g_nc, iou, 0.0)
        mx = jnp.max(ov_nc)
        asn = jnp.min(jnp.where(ov_nc == mx, g_iota, jnp.int32(2 ** 30)))
        sel = g_iota == asn
        by1 = jnp.sum(jnp.where(sel, gy1, 0.0))
        bx1 = jnp.sum(jnp.where(sel, gx1, 0.0))
        by2 = jnp.sum(jnp.where(sel, gy2, 0.0))
        bx2 = jnp.sum(jnp.where(sel, gx2, 0.0))
        cid = jnp.sum(jnp.where(sel, gids, 0))
        rh = jnp.maximum((py2 - py1) * vf, 1e-8)
        rw = jnp.maximum((px2 - px1) * vf, 1e-8)
        rcy = py1 * vf + 0.5 * rh
        rcx = px1 * vf + 0.5 * rw
        gh = jnp.maximum(by2 - by1, 1e-8)
        gw = jnp.maximum(bx2 - bx1, 1e-8)
        gcy = by1 + 0.5 * gh
        gcx = bx1 + 0.5 * gw
        logs = jnp.log(jnp.where(lane4 == 2, gh / rh,
                                 jnp.where(lane4 == 3, gw / rw, 1.0)))
        dyv = (gcy - rcy) / rh
        dxv = (gcx - rcx) / rw
        drow = jnp.where(lane4 == 0, dyv,
                         jnp.where(lane4 == 1, dxv, logs))
        drow = (drow / std_row) * vf
        deltas_ref[pl.ds(i, 1), :] = drow
        ids_ref[pl.ds(i, 1), :] = jnp.full((1, 1), cid * valid.astype(jnp.int32),
                                           jnp.int32)
        assign_ref[pl.ds(i, 1), :] = jnp.full((1, 1), asn, jnp.int32)
        pvalid_ref[pl.ds(i, 1), :] = jnp.full((1, 1), vf, f32)
        return jnp.where(flat_iota == idx, -jnp.inf, scores)

    jax.lax.fori_loop(0, _POS, pos_step, pos_scores)

    def neg_step(i, scores):
        m = jnp.max(scores)
        idx = jnp.min(jnp.where(scores == m, flat_iota, jnp.int32(2 ** 30)))
        vf = (m > 0.0).astype(f32)
        row = props_ref[pl.ds(idx, 1), :]
        j = i + _POS
        rois_ref[pl.ds(j, 1), :] = row * vf
        deltas_ref[pl.ds(j, 1), :] = jnp.zeros((1, 4), f32)
        ids_ref[pl.ds(j, 1), :] = jnp.zeros((1, 1), jnp.int32)
        assign_ref[pl.ds(j, 1), :] = jnp.zeros((1, 1), jnp.int32)
        pvalid_ref[pl.ds(j, 1), :] = jnp.zeros((1, 1), f32)
        return jnp.where(flat_iota == idx, -jnp.inf, scores)

    jax.lax.fori_loop(0, _NEG, neg_step, neg_scores)


def _crop_kernel(assign_s_ref, mask_ref, rois_s_ref, pv_s_ref, out_ref):
    f32 = jnp.float32
    i = pl.program_id(0)
    m = mask_ref[0].astype(f32)
    y1 = rois_s_ref[i, 0]
    x1 = rois_s_ref[i, 1]
    y2 = rois_s_ref[i, 2]
    x2 = rois_s_ref[i, 3]
    pv = pv_s_ref[i]

    iy = jax.lax.broadcasted_iota(jnp.int32, (_MH, 1), 0).astype(f32)
    ys = y1 * (_HM - 1) + iy * ((y2 - y1) * (_HM - 1) / (_MH - 1))
    y0 = jnp.floor(ys)
    y0i = jnp.clip(y0, 0, _HM - 1).astype(jnp.int32)
    y1i = jnp.clip(y0 + 1, 0, _HM - 1).astype(jnp.int32)
    wy = ys - y0

    ixc = jax.lax.broadcasted_iota(jnp.int32, (_MW, 1), 0).astype(f32)
    xs_c = x1 * (_WM - 1) + ixc * ((x2 - x1) * (_WM - 1) / (_MW - 1))
    x0_c = jnp.floor(xs_c)
    x0i = jnp.clip(x0_c, 0, _WM - 1).astype(jnp.int32)
    x1i = jnp.clip(x0_c + 1, 0, _WM - 1).astype(jnp.int32)

    ixl = jax.lax.broadcasted_iota(jnp.int32, (1, _MW), 1).astype(f32)
    xs_l = x1 * (_WM - 1) + ixl * ((x2 - x1) * (_WM - 1) / (_MW - 1))
    wx = xs_l - jnp.floor(xs_l)

    ycat = jnp.concatenate([y0i, y1i], axis=0)
    xcat = jnp.concatenate([x0i, x1i], axis=0)
    lane_h = jax.lax.broadcasted_iota(jnp.int32, (2 * _MH, _HM), 1)
    Ycat = (lane_h == ycat).astype(f32)
    lane_w = jax.lax.broadcasted_iota(jnp.int32, (2 * _MW, _WM), 1)
    Xcat = (lane_w == xcat).astype(f32)

    dn = (((1,), (1,)), ((), ()))
    dnm = (((1,), (0,)), ((), ()))
    Acat = jax.lax.dot_general(Ycat, m, dnm, preferred_element_type=f32)
    G = jax.lax.dot_general(Acat, Xcat, dn, preferred_element_type=f32)
    g00 = G[:_MH, :_MW]
    g01 = G[:_MH, _MW:]
    g10 = G[_MH:, :_MW]
    g11 = G[_MH:, _MW:]

    val = (((1.0 - wy) * (1.0 - wx)) * g00
           + ((1.0 - wy) * wx) * g01
           + (wy * (1.0 - wx)) * g10
           + (wy * wx) * g11)
    in_y = jnp.logical_and(ys >= 0.0, ys <= (_HM - 1))
    in_x = jnp.logical_and(xs_l >= 0.0, xs_l <= (_WM - 1))
    val = jnp.where(jnp.logical_and(in_y, in_x), val, 0.0)
    out_ref[0] = jnp.round(val) * pv


@jax.jit
def kernel(proposals, prior_class_ids, prior_boxes, prior_masks):
    f32 = jnp.float32
    props = proposals[0]
    props_pad = jnp.pad(props, ((0, _N_PAD - _N_PROP), (0, 0)))
    coords = jnp.reshape(jnp.transpose(props_pad), (4, _ROWS, _COLS))
    y1p, x1p, y2p, x2p = coords[0], coords[1], coords[2], coords[3]

    gtb = jnp.pad(prior_boxes[0], ((0, _GT_PAD - _MAX_GT), (0, 0)))
    gtb_t = jnp.transpose(gtb)
    ids_row = jnp.pad(prior_class_ids[0], (0, _GT_PAD - _MAX_GT))[None, :]

    smem = pl.BlockSpec(memory_space=pltpu.SMEM)
    rois, out_ids, deltas, assign, pvalid = pl.pallas_call(
        _select_kernel,
        in_specs=[pl.BlockSpec(props_pad.shape, lambda: (0, 0)),
                  pl.BlockSpec((_ROWS, _COLS), lambda: (0, 0)),
                  pl.BlockSpec((_ROWS, _COLS), lambda: (0, 0)),
                  pl.BlockSpec((_ROWS, _COLS), lambda: (0, 0)),
                  pl.BlockSpec((_ROWS, _COLS), lambda: (0, 0)),
                  smem, smem,
                  pl.BlockSpec((4, _GT_PAD), lambda: (0, 0)),
                  pl.BlockSpec((1, _GT_PAD), lambda: (0, 0))],
        out_specs=[pl.BlockSpec((_TOT, 4), lambda: (0, 0)),
                   pl.BlockSpec((_TOT, 1), lambda: (0, 0)),
                   pl.BlockSpec((_TOT, 4), lambda: (0, 0)),
                   pl.BlockSpec((_TOT, 1), lambda: (0, 0)),
                   pl.BlockSpec((_TOT, 1), lambda: (0, 0))],
        out_shape=[jax.ShapeDtypeStruct((_TOT, 4), f32),
                   jax.ShapeDtypeStruct((_TOT, 1), jnp.int32),
                   jax.ShapeDtypeStruct((_TOT, 4), f32),
                   jax.ShapeDtypeStruct((_TOT, 1), jnp.int32),
                   jax.ShapeDtypeStruct((_TOT, 1), f32)],
    )(props_pad, y1p, x1p, y2p, x2p, gtb_t, ids_row, gtb_t, ids_row)

    masks_t = jnp.transpose(prior_masks[0], (2, 0, 1)).astype(jnp.int8)
    assign_flat = assign[:_POS, 0]
    pos_rois = rois[:_POS]
    pv_flat = pvalid[:_POS, 0]

    grid_spec = pltpu.PrefetchScalarGridSpec(
        num_scalar_prefetch=1,
        grid=(_POS,),
        in_specs=[pl.BlockSpec((1, _HM, _WM), lambda i, a: (a[i], 0, 0)),
                  pl.BlockSpec(memory_space=pltpu.SMEM),
                  pl.BlockSpec(memory_space=pltpu.SMEM)],
        out_specs=pl.BlockSpec((1, _MH, _MW), lambda i, a: (i, 0, 0)),
    )
    cropped = pl.pallas_call(
        _crop_kernel,
        grid_spec=grid_spec,
        out_shape=jax.ShapeDtypeStruct((_POS, _MH, _MW), f32),
    )(assign_flat, masks_t, pos_rois, pv_flat)

    out_masks = jnp.concatenate(
        [cropped, jnp.zeros((_NEG, _MH, _MW), f32)], axis=0)
    return (rois[None], out_ids[None, :, 0], deltas[None],
            out_masks[None])
